# Initial kernel scaffold; baseline (speedup 1.0000x reference)
import jax
import jax.numpy as jnp
from jax.experimental import pallas as pl


def kernel(x, W):
    raise NotImplementedError("write your pallas kernel here")


# speedup vs baseline: 8.4168x; 8.4168x over previous
import jax
import jax.numpy as jnp
from jax.experimental import pallas as pl
from jax.experimental.pallas import tpu as pltpu

CHANNEL_IN = 256
CHANNEL_OUT = 32768
GROUP = 8
BATCH = 128

COL_BLK = 2048


def _fused_kernel(x_ref, w_ref, o_ref):
    y = jnp.dot(x_ref[...], w_ref[...], preferred_element_type=jnp.float32)
    b, n = y.shape
    y3 = y.reshape(b, n // GROUP, GROUP)
    idx = jnp.argmax(y3, axis=2)
    iota = jax.lax.broadcasted_iota(jnp.int32, (b, n // GROUP, GROUP), 2)
    onehot = (iota == idx[:, :, None]).astype(jnp.float32)
    o_ref[...] = onehot.reshape(b, n)


def kernel(x, W):
    grid = (CHANNEL_OUT // COL_BLK,)
    return pl.pallas_call(
        _fused_kernel,
        grid=grid,
        in_specs=[
            pl.BlockSpec((BATCH, CHANNEL_IN), lambda j: (0, 0)),
            pl.BlockSpec((CHANNEL_IN, COL_BLK), lambda j: (0, j)),
        ],
        out_specs=pl.BlockSpec((BATCH, COL_BLK), lambda j: (0, j)),
        out_shape=jax.ShapeDtypeStruct((BATCH, CHANNEL_OUT), jnp.float32),
        compiler_params=pltpu.CompilerParams(
            dimension_semantics=("arbitrary",),
        ),
    )(x, W)


# speedup vs baseline: 16.8146x; 1.9978x over previous
import jax
import jax.numpy as jnp
from jax.experimental import pallas as pl
from jax.experimental.pallas import tpu as pltpu

CHANNEL_IN = 256
CHANNEL_OUT = 32768
GROUP = 8
BATCH = 128

COL_BLK = 2048
CHUNK = 128


def _fused_kernel(x_ref, w_ref, o_ref):
    y = jnp.dot(x_ref[...], w_ref[...], preferred_element_type=jnp.float32)
    q = jax.lax.broadcasted_iota(jnp.int32, (BATCH, CHUNK), 1) % GROUP
    masks = [q < (GROUP - k) for k in (1, 2, 4)]
    for j in range(COL_BLK // CHUNK):
        yj = y[:, j * CHUNK:(j + 1) * CHUNK]
        v = yj
        for k, m in zip((1, 2, 4), masks):
            vr = jnp.where(m, pltpu.roll(v, CHUNK - k, 1), pltpu.roll(v, GROUP - k, 1))
            v = jnp.maximum(v, vr)
        c = jnp.where(yj == v, q, GROUP)
        for k, m in zip((1, 2, 4), masks):
            cr = jnp.where(m, pltpu.roll(c, CHUNK - k, 1), pltpu.roll(c, GROUP - k, 1))
            c = jnp.minimum(c, cr)
        o_ref[:, j * CHUNK:(j + 1) * CHUNK] = (q == c).astype(jnp.float32)


def kernel(x, W):
    grid = (CHANNEL_OUT // COL_BLK,)
    return pl.pallas_call(
        _fused_kernel,
        grid=grid,
        in_specs=[
            pl.BlockSpec((BATCH, CHANNEL_IN), lambda j: (0, 0)),
            pl.BlockSpec((CHANNEL_IN, COL_BLK), lambda j: (0, j)),
        ],
        out_specs=pl.BlockSpec((BATCH, COL_BLK), lambda j: (0, j)),
        out_shape=jax.ShapeDtypeStruct((BATCH, CHANNEL_OUT), jnp.float32),
        compiler_params=pltpu.CompilerParams(
            dimension_semantics=("arbitrary",),
        ),
    )(x, W)


# speedup vs baseline: 24.6120x; 1.4637x over previous
import jax
import jax.numpy as jnp
from jax.experimental import pallas as pl
from jax.experimental.pallas import tpu as pltpu

CHANNEL_IN = 256
CHANNEL_OUT = 32768
GROUP = 8
BATCH = 128

COL_BLK = 2048
CHUNK = 128


def _fused_kernel(x_ref, w_ref, o_ref):
    x = x_ref[...]
    q = jax.lax.broadcasted_iota(jnp.int32, (BATCH, CHUNK), 1) % GROUP
    masks = [q < (GROUP - k) for k in (1, 2, 4)]
    for j in range(COL_BLK // CHUNK):
        yj = jnp.dot(x, w_ref[:, j * CHUNK:(j + 1) * CHUNK],
                     preferred_element_type=jnp.float32)
        v, c = yj, q
        for k, m in zip((1, 2, 4), masks):
            vr = jnp.where(m, pltpu.roll(v, CHUNK - k, 1), pltpu.roll(v, GROUP - k, 1))
            cr = jnp.where(m, pltpu.roll(c, CHUNK - k, 1), pltpu.roll(c, GROUP - k, 1))
            take = (vr > v) | ((vr == v) & (cr < c))
            v = jnp.where(take, vr, v)
            c = jnp.where(take, cr, c)
        o_ref[:, j * CHUNK:(j + 1) * CHUNK] = (q == c).astype(jnp.float32)


def kernel(x, W):
    grid = (CHANNEL_OUT // COL_BLK,)
    return pl.pallas_call(
        _fused_kernel,
        grid=grid,
        in_specs=[
            pl.BlockSpec((BATCH, CHANNEL_IN), lambda j: (0, 0)),
            pl.BlockSpec((CHANNEL_IN, COL_BLK), lambda j: (0, j)),
        ],
        out_specs=pl.BlockSpec((BATCH, COL_BLK), lambda j: (0, j)),
        out_shape=jax.ShapeDtypeStruct((BATCH, CHANNEL_OUT), jnp.float32),
        compiler_params=pltpu.CompilerParams(
            dimension_semantics=("arbitrary",),
        ),
    )(x, W)


# speedup vs baseline: 61.3410x; 2.4923x over previous
import jax
import jax.numpy as jnp
from jax.experimental import pallas as pl
from jax.experimental.pallas import tpu as pltpu

CHANNEL_IN = 256
CHANNEL_OUT = 32768
GROUP = 8
BATCH = 128

COL_BLK = 2048


def _fused_kernel(xt_ref, w_ref, o_ref):
    yt = jax.lax.dot_general(
        w_ref[...], xt_ref[...], (((0,), (0,)), ((), ())),
        preferred_element_type=jnp.float32)
    y3 = yt.reshape(COL_BLK // GROUP, GROUP, BATCH)
    s = jax.lax.broadcasted_iota(jnp.int32, (COL_BLK // GROUP, GROUP, BATCH), 1)
    v, c = y3, s
    for k in (1, 2, 4):
        vr = pltpu.roll(v, GROUP - k, 1)
        cr = pltpu.roll(c, GROUP - k, 1)
        take = (vr > v) | ((vr == v) & (cr < c))
        v = jnp.where(take, vr, v)
        c = jnp.where(take, cr, c)
    onehot = (s == c).astype(jnp.float32).reshape(COL_BLK, BATCH)
    o_ref[...] = onehot.T


def kernel(x, W):
    grid = (CHANNEL_OUT // COL_BLK,)
    return pl.pallas_call(
        _fused_kernel,
        grid=grid,
        in_specs=[
            pl.BlockSpec((CHANNEL_IN, BATCH), lambda j: (0, 0)),
            pl.BlockSpec((CHANNEL_IN, COL_BLK), lambda j: (0, j)),
        ],
        out_specs=pl.BlockSpec((BATCH, COL_BLK), lambda j: (0, j)),
        out_shape=jax.ShapeDtypeStruct((BATCH, CHANNEL_OUT), jnp.float32),
        compiler_params=pltpu.CompilerParams(
            dimension_semantics=("arbitrary",),
        ),
    )(x.T, W)


# speedup vs baseline: 70.9153x; 1.1561x over previous
import jax
import jax.numpy as jnp
from jax.experimental import pallas as pl
from jax.experimental.pallas import tpu as pltpu

CHANNEL_IN = 256
CHANNEL_OUT = 32768
GROUP = 8
BATCH = 128

COL_BLK = 4096


def _fused_kernel(xt_ref, w_ref, o_ref):
    yt = jax.lax.dot_general(
        w_ref[...], xt_ref[...], (((0,), (0,)), ((), ())),
        preferred_element_type=jnp.float32)
    y3 = yt.reshape(COL_BLK // GROUP, GROUP, BATCH)
    s = jax.lax.broadcasted_iota(jnp.int32, (COL_BLK // GROUP, GROUP, BATCH), 1)
    v, c = y3, s
    for k in (1, 2, 4):
        vr = pltpu.roll(v, GROUP - k, 1)
        cr = pltpu.roll(c, GROUP - k, 1)
        take = (vr > v) | ((vr == v) & (cr < c))
        v = jnp.where(take, vr, v)
        c = jnp.where(take, cr, c)
    onehot = (s == c).astype(jnp.float32).reshape(COL_BLK, BATCH)
    o_ref[...] = onehot.T


def kernel(x, W):
    grid = (CHANNEL_OUT // COL_BLK,)
    return pl.pallas_call(
        _fused_kernel,
        grid=grid,
        in_specs=[
            pl.BlockSpec((CHANNEL_IN, BATCH), lambda j: (0, 0)),
            pl.BlockSpec((CHANNEL_IN, COL_BLK), lambda j: (0, j)),
        ],
        out_specs=pl.BlockSpec((BATCH, COL_BLK), lambda j: (0, j)),
        out_shape=jax.ShapeDtypeStruct((BATCH, CHANNEL_OUT), jnp.float32),
        compiler_params=pltpu.CompilerParams(
            dimension_semantics=("arbitrary",),
        ),
    )(x.T, W)


# speedup vs baseline: 72.3100x; 1.0197x over previous
import jax
import jax.numpy as jnp
from jax.experimental import pallas as pl
from jax.experimental.pallas import tpu as pltpu

CHANNEL_IN = 256
CHANNEL_OUT = 32768
GROUP = 8
BATCH = 128

COL_BLK = 8192


def _fused_kernel(xt_ref, w_ref, o_ref):
    yt = jax.lax.dot_general(
        w_ref[...], xt_ref[...], (((0,), (0,)), ((), ())),
        preferred_element_type=jnp.float32)
    y3 = yt.reshape(COL_BLK // GROUP, GROUP, BATCH)
    s = jax.lax.broadcasted_iota(jnp.int32, (COL_BLK // GROUP, GROUP, BATCH), 1)
    v, c = y3, s
    for k in (1, 2, 4):
        vr = pltpu.roll(v, GROUP - k, 1)
        cr = pltpu.roll(c, GROUP - k, 1)
        take = (vr > v) | ((vr == v) & (cr < c))
        v = jnp.where(take, vr, v)
        c = jnp.where(take, cr, c)
    onehot = (s == c).astype(jnp.float32).reshape(COL_BLK, BATCH)
    o_ref[...] = onehot.T


def kernel(x, W):
    grid = (CHANNEL_OUT // COL_BLK,)
    return pl.pallas_call(
        _fused_kernel,
        grid=grid,
        in_specs=[
            pl.BlockSpec((CHANNEL_IN, BATCH), lambda j: (0, 0)),
            pl.BlockSpec((CHANNEL_IN, COL_BLK), lambda j: (0, j)),
        ],
        out_specs=pl.BlockSpec((BATCH, COL_BLK), lambda j: (0, j)),
        out_shape=jax.ShapeDtypeStruct((BATCH, CHANNEL_OUT), jnp.float32),
        compiler_params=pltpu.CompilerParams(
            dimension_semantics=("arbitrary",),
        ),
    )(x.T, W)
